# baseline (speedup 1.0000x reference)
import jax
import jax.numpy as jnp
from jax import lax
from jax.experimental import pallas as pl
from jax.experimental.pallas import tpu as pltpu
from jax.experimental.pallas import tpu_sc as plsc

V = 1000000
D = 64
B = 16384
K = 20

NC = 2
NS = 16
L = 16
NW = NC * NS

ITEMS_PER_W = B // NW
CHUNK = 64
NCHUNK = ITEMS_PER_W // CHUNK
KRES = 10
UNROLL = 8
BLK = (K + 1) * CHUNK


def _sc_scores_body(w_in, w_out, center, context, neg_flat, out, *refs):
    (idx_c, idx_x, negs_v, idx_all, rows_c, rows_x, sc_all) = refs[:7]
    nbuf = (refs[7:7 + KRES], refs[7 + KRES:7 + 2 * KRES])
    sem_c, sem_x, sem_r0, sem_r1 = refs[7 + 2 * KRES:]
    sem_r = (sem_r0, sem_r1)
    w = lax.axis_index("s") * NC + lax.axis_index("c")
    iota = lax.iota(jnp.int32, L)

    def stage_and_issue(c):
        half = c % 2
        base = w * ITEMS_PER_W + c * CHUNK
        pltpu.sync_copy(center.at[pl.ds(base, CHUNK)],
                        idx_c.at[pl.ds(half * CHUNK, CHUNK)])
        pltpu.sync_copy(context.at[pl.ds(base, CHUNK)],
                        idx_x.at[pl.ds(half * CHUNK, CHUNK)])
        pltpu.async_copy(w_in.at[idx_c.at[pl.ds(half * CHUNK, CHUNK)]],
                         rows_c.at[pl.ds(half * CHUNK, CHUNK), :], sem_c)
        pltpu.async_copy(w_out.at[idx_x.at[pl.ds(half * CHUNK, CHUNK)]],
                         rows_x.at[pl.ds(half * CHUNK, CHUNK), :], sem_x)
        noff = half * (CHUNK * K)
        pltpu.sync_copy(neg_flat.at[pl.ds(base * K, CHUNK * K)],
                        negs_v.at[pl.ds(noff, CHUNK * K)])
        for k in range(K):
            def j_body(j, _, k=k):
                lanes = iota * K + (noff + j * (L * K) + k)
                idx_all[pl.ds(noff + k * CHUNK + j * L, L)] = (
                    plsc.load_gather(negs_v, [lanes]))
                return 0

            lax.fori_loop(0, CHUNK // L, j_body, 0)
        issue_round(c, 0)

    def issue_round(c, r):
        noff = (c % 2) * (CHUNK * K)
        for kk in range(KRES):
            pltpu.async_copy(
                w_out.at[idx_all.at[pl.ds(noff + (r * KRES + kk) * CHUNK,
                                          CHUNK)]],
                nbuf[r][kk], sem_r[r])

    def drain(sem, nbytes_rows):
        def body(i, _):
            pltpu.make_async_copy(w_out.at[pl.ds(0, CHUNK), :],
                                  nbuf[0][0], sem).wait()
            return 0

        lax.fori_loop(0, nbytes_rows // CHUNK, body, 0)

    def pos_compute(c):
        half = c % 2

        def group(g, _):
            row = iota + (half * CHUNK + g * L)

            def dstep(t, accs):
                a0, a1 = accs
                for u in range(UNROLL // 2):
                    dd = t * UNROLL + 2 * u
                    c0 = jnp.full((L,), dd, jnp.int32)
                    c1 = jnp.full((L,), dd + 1, jnp.int32)
                    a0 = a0 + (plsc.load_gather(rows_c, [row, c0])
                               * plsc.load_gather(rows_x, [row, c0]))
                    a1 = a1 + (plsc.load_gather(rows_c, [row, c1])
                               * plsc.load_gather(rows_x, [row, c1]))
                return (a0, a1)

            z = jnp.zeros((L,), jnp.float32)
            a0, a1 = lax.fori_loop(0, D // UNROLL, dstep, (z, z))
            sc_all[pl.ds(g * L, L)] = a0 + a1
            return 0

        lax.fori_loop(0, CHUNK // L, group, 0)

    def round_compute(c, r):
        half = c % 2
        bufs = nbuf[r]

        def group(g, _):
            row = iota + (half * CHUNK + g * L)
            rowb = iota + g * L

            def dstep(t, accs):
                for u in range(UNROLL):
                    col = jnp.full((L,), t * UNROLL + u, jnp.int32)
                    cvec = plsc.load_gather(rows_c, [row, col])
                    accs = tuple(
                        accs[kk] + cvec * plsc.load_gather(bufs[kk],
                                                           [rowb, col])
                        for kk in range(KRES))
                return accs

            z = jnp.zeros((L,), jnp.float32)
            accs = lax.fori_loop(0, D // UNROLL, dstep, (z,) * KRES)
            for kk in range(KRES):
                sc_all[pl.ds((1 + r * KRES + kk) * CHUNK + g * L, L)] = (
                    -accs[kk])
            return 0

        lax.fori_loop(0, CHUNK // L, group, 0)

    def chunk_body(c, _):
        drain(sem_c, CHUNK)
        drain(sem_x, CHUNK)
        pos_compute(c)
        issue_round(c, 1)
        drain(sem_r0, KRES * CHUNK)
        round_compute(c, 0)

        @pl.when(c < NCHUNK - 1)
        def _():
            stage_and_issue(c + 1)

        drain(sem_r1, KRES * CHUNK)
        round_compute(c, 1)
        pltpu.sync_copy(sc_all,
                        out.at[pl.ds((w * NCHUNK + c) * BLK, BLK)])
        return 0

    stage_and_issue(0)
    lax.fori_loop(0, NCHUNK, chunk_body, 0)


def _sc_scores(w_in, w_out, center, context, neg_flat):
    mesh = plsc.VectorSubcoreMesh(core_axis_name="c", subcore_axis_name="s",
                                  num_cores=NC, num_subcores=NS)
    scratch = [
        pltpu.VMEM((2 * CHUNK,), jnp.int32),
        pltpu.VMEM((2 * CHUNK,), jnp.int32),
        pltpu.VMEM((2 * CHUNK * K,), jnp.int32),
        pltpu.VMEM((2 * CHUNK * K,), jnp.int32),
        pltpu.VMEM((2 * CHUNK, D), jnp.float32),
        pltpu.VMEM((2 * CHUNK, D), jnp.float32),
        pltpu.VMEM((BLK,), jnp.float32),
    ]
    scratch += [pltpu.VMEM((CHUNK, D), jnp.float32)
                for _ in range(2 * KRES)]
    scratch += [pltpu.SemaphoreType.DMA] * 4
    fn = pl.kernel(
        _sc_scores_body,
        out_type=jax.ShapeDtypeStruct((NW * NCHUNK * BLK,), jnp.float32),
        mesh=mesh,
        compiler_params=pltpu.CompilerParams(
            needs_layout_passes=False, use_tc_tiling_on_sc=False),
        scratch_types=scratch,
    )
    return fn(w_in, w_out, center, context, neg_flat)


def _loss_body(s_ref, o_ref):
    x = s_ref[...]
    losses = -jnp.log(jax.nn.sigmoid(x) + 1e-10)
    o_ref[...] = jnp.reshape(jnp.sum(losses) * (1.0 / B), (1, 1))


def _loss(scores2d):
    out = pl.pallas_call(
        _loss_body,
        out_shape=jax.ShapeDtypeStruct((1, 1), jnp.float32),
    )(scores2d)
    return out[0, 0]


def kernel(center, context, negatives, W_in, W_out):
    center = center.astype(jnp.int32)
    context = context.astype(jnp.int32)
    neg_flat = negatives.astype(jnp.int32).reshape(B * K)
    scores = _sc_scores(W_in, W_out, center, context, neg_flat)
    return _loss(scores.reshape((K + 1) * B // 128, 128))
